# baseline (speedup 1.0000x reference)
import functools

import jax
import jax.numpy as jnp
from jax import lax
from jax.experimental import pallas as pl
from jax.experimental.pallas import tpu as pltpu
from jax.experimental.pallas import tpu_sc as plsc

B = 16384
D = 64
NW = 32
BPW = B // NW
BLK = 64
NBLK = BPW // BLK
L = 16


def _fold_body(w1, w2, w3, wp, b1, b2, b3, bp, w_out, c_out):
    wpa = wp[0:32, :]
    v3 = jnp.dot(w3[...], wpa, preferred_element_type=jnp.float32)
    v2 = jnp.dot(w2[...], v3, preferred_element_type=jnp.float32)
    w = jnp.dot(w1[...], v2, preferred_element_type=jnp.float32)
    c = (jnp.sum(b1[...] * v2) + jnp.sum(b2[...] * v3)
         + jnp.sum(b3[...] * wpa) + bp[0, 0])
    w_out[...] = w
    c_out[...] = jnp.broadcast_to(c, (1, 1))


def _fold(W1, b1, W2, b2, W3, b3, Wp, bp):
    return pl.pallas_call(
        _fold_body,
        out_shape=(
            jax.ShapeDtypeStruct((128, 1), jnp.float32),
            jax.ShapeDtypeStruct((1, 1), jnp.float32),
        ),
    )(W1, W2, W3, Wp, b1.reshape(128, 1), b2.reshape(64, 1),
      b3.reshape(32, 1), bp.reshape(1, 1))


def _sc_body(user, item, tab_mu, tab_fu, tab_mi, tab_fi, wpack, out,
             idxu, idxi, rmu, rmi, rfu, rfi, wv, outbuf,
             sem0, sem1, sem2, sem3):
    wid = lax.axis_index("s") * 2 + lax.axis_index("c")
    base = wid * BPW

    pltpu.sync_copy(wpack, wv)
    wu = [wv[0, pl.ds(k * L, L)] for k in range(4)]
    wi = [wv[0, pl.ds(64 + k * L, L)] for k in range(4)]
    wb = [wv[1, pl.ds(k * L, L)] for k in range(4)]
    cvec = wv[1, pl.ds(64, L)]

    pltpu.sync_copy(user.at[pl.ds(base, BPW)], idxu)
    pltpu.sync_copy(item.at[pl.ds(base, BPW)], idxi)

    rowids = lax.iota(jnp.int32, L)
    perms = {sh: rowids ^ sh for sh in (8, 4, 2, 1)}

    def block(g, carry):
        hs = []
        for q in range(BLK // L):
            uvals = idxu[pl.ds(g * BLK + q * L, L)]
            ivals = idxi[pl.ds(g * BLK + q * L, L)]
            for r in range(L):
                u = uvals[r]
                it = ivals[r]
                dst = pl.ds(q * L + r, 1)
                hs.append(pltpu.async_copy(
                    tab_mu.at[pl.ds(u, 1), :], rmu.at[dst, :], sem0))
                hs.append(pltpu.async_copy(
                    tab_fu.at[pl.ds(u, 1), :], rfu.at[dst, :], sem1))
                hs.append(pltpu.async_copy(
                    tab_mi.at[pl.ds(it, 1), :], rmi.at[dst, :], sem2))
                hs.append(pltpu.async_copy(
                    tab_fi.at[pl.ds(it, 1), :], rfi.at[dst, :], sem3))
        for h in hs:
            h.wait()
        for q in range(BLK // L):
            s = cvec
            for r in range(L):
                row = q * L + r
                acc = rmu[row, pl.ds(0, L)] * wu[0]
                for k in range(1, 4):
                    acc = acc + rmu[row, pl.ds(k * L, L)] * wu[k]
                for k in range(4):
                    acc = acc + rmi[row, pl.ds(k * L, L)] * wi[k]
                for k in range(4):
                    acc = acc + (rfu[row, pl.ds(k * L, L)]
                                 * rfi[row, pl.ds(k * L, L)]) * wb[k]
                for sh in (8, 4, 2, 1):
                    acc = acc + acc.at[perms[sh]].get(
                        mode="promise_in_bounds")
                s = jnp.where(rowids == r, acc, s)
            o = 1.0 / (1.0 + jnp.exp(-s))
            outbuf[pl.ds(g * BLK + q * L, L)] = o
        return carry

    lax.fori_loop(0, NBLK, block, 0)
    pltpu.sync_copy(outbuf, out.at[pl.ds(base, BPW)])


@functools.partial(jax.jit, static_argnums=())
def _sc_forward(user, item, tab_mu, tab_fu, tab_mi, tab_fi, wpack):
    mesh = plsc.VectorSubcoreMesh(core_axis_name="c", subcore_axis_name="s")
    scratch = (
        pltpu.VMEM((BPW,), jnp.int32),
        pltpu.VMEM((BPW,), jnp.int32),
        pltpu.VMEM((BLK, D), jnp.float32),
        pltpu.VMEM((BLK, D), jnp.float32),
        pltpu.VMEM((BLK, D), jnp.float32),
        pltpu.VMEM((BLK, D), jnp.float32),
        pltpu.VMEM((2, 128), jnp.float32),
        pltpu.VMEM((BPW,), jnp.float32),
        pltpu.SemaphoreType.DMA,
        pltpu.SemaphoreType.DMA,
        pltpu.SemaphoreType.DMA,
        pltpu.SemaphoreType.DMA,
    )
    f = pl.kernel(
        _sc_body,
        out_type=jax.ShapeDtypeStruct((B,), jnp.float32),
        mesh=mesh,
        scratch_types=scratch,
    )
    return f(user, item, tab_mu, tab_fu, tab_mi, tab_fi, wpack)


def kernel(user, item, mlp_user_table, mf_user_table, mlp_item_table,
           mf_item_table, W1, b1, W2, b2, W3, b3, Wp, bp):
    w2d, c2d = _fold(W1, b1, W2, b2, W3, b3, Wp, bp)
    row1 = jnp.concatenate(
        [Wp[32:, 0], jnp.broadcast_to(c2d[0, 0], (64,))])
    wpack = jnp.stack([w2d[:, 0], row1])
    out = _sc_forward(user.astype(jnp.int32), item.astype(jnp.int32),
                      mlp_user_table, mf_user_table, mlp_item_table,
                      mf_item_table, wpack)
    return out.reshape(B, 1)


# speedup vs baseline: 1.0142x; 1.0142x over previous
import functools

import jax
import jax.numpy as jnp
from jax import lax
from jax.experimental import pallas as pl
from jax.experimental.pallas import tpu as pltpu
from jax.experimental.pallas import tpu_sc as plsc

B = 16384
D = 64
NW = 32
BPW = B // NW
HALF = 128
NH = BPW // HALF
L = 16


def _fold_body(w1, w2, w3, wp, b1, b2, b3, bp, w_out, c_out):
    wpa = wp[0:32, :]
    v3 = jnp.dot(w3[...], wpa, preferred_element_type=jnp.float32)
    v2 = jnp.dot(w2[...], v3, preferred_element_type=jnp.float32)
    w = jnp.dot(w1[...], v2, preferred_element_type=jnp.float32)
    c = (jnp.sum(b1[...] * v2) + jnp.sum(b2[...] * v3)
         + jnp.sum(b3[...] * wpa) + bp[0, 0])
    w_out[...] = w
    c_out[...] = jnp.broadcast_to(c, (1, 1))


def _fold(W1, b1, W2, b2, W3, b3, Wp, bp):
    return pl.pallas_call(
        _fold_body,
        out_shape=(
            jax.ShapeDtypeStruct((128, 1), jnp.float32),
            jax.ShapeDtypeStruct((1, 1), jnp.float32),
        ),
    )(W1, W2, W3, Wp, b1.reshape(128, 1), b2.reshape(64, 1),
      b3.reshape(32, 1), bp.reshape(1, 1))


def _sc_body(user, item, tab_mu, tab_fu, tab_mi, tab_fi, wpack, out,
             idxu, idxi, rmu, rmi, rfu, rfi, wv, outbuf,
             sem0, sem1, sem2, sem3):
    wid = lax.axis_index("s") * 2 + lax.axis_index("c")
    base = wid * BPW

    pltpu.sync_copy(wpack, wv)
    wu = [wv[0, pl.ds(k * L, L)] for k in range(4)]
    wi = [wv[0, pl.ds(64 + k * L, L)] for k in range(4)]
    wb = [wv[1, pl.ds(k * L, L)] for k in range(4)]
    cvec = wv[1, pl.ds(64, L)]

    pltpu.sync_copy(user.at[pl.ds(base, BPW)], idxu)
    pltpu.sync_copy(item.at[pl.ds(base, BPW)], idxi)

    rowids = lax.iota(jnp.int32, L)
    perms = {sh: rowids ^ sh for sh in (8, 4, 2, 1)}

    def fire(g, carry, h=None):
        uvals = idxu[pl.ds(h * HALF + g * L, L)]
        ivals = idxi[pl.ds(h * HALF + g * L, L)]
        for r in range(L):
            u = uvals[r]
            it = ivals[r]
            dst = pl.ds(g * L + r, 1)
            pltpu.async_copy(tab_mu.at[pl.ds(u, 1), :], rmu.at[dst, :], sem0)
            pltpu.async_copy(tab_fu.at[pl.ds(u, 1), :], rfu.at[dst, :], sem1)
            pltpu.async_copy(tab_mi.at[pl.ds(it, 1), :], rmi.at[dst, :], sem2)
            pltpu.async_copy(tab_fi.at[pl.ds(it, 1), :], rfi.at[dst, :], sem3)
        return carry

    def compute(g, carry, h=None):
        s = cvec
        for r in range(L):
            row = g * L + r
            acc = rmu[row, pl.ds(0, L)] * wu[0]
            for k in range(1, 4):
                acc = acc + rmu[row, pl.ds(k * L, L)] * wu[k]
            for k in range(4):
                acc = acc + rmi[row, pl.ds(k * L, L)] * wi[k]
            for k in range(4):
                acc = acc + (rfu[row, pl.ds(k * L, L)]
                             * rfi[row, pl.ds(k * L, L)]) * wb[k]
            for sh in (8, 4, 2, 1):
                acc = acc + acc.at[perms[sh]].get(mode="promise_in_bounds")
            s = jnp.where(rowids == r, acc, s)
        o = 1.0 / (1.0 + jnp.exp(-s))
        outbuf[pl.ds(h * HALF + g * L, L)] = o
        return carry

    for h in range(NH):
        lax.fori_loop(0, HALF // L, functools.partial(fire, h=h), 0)
        pltpu.make_async_copy(tab_mu.at[pl.ds(0, HALF), :], rmu, sem0).wait()
        pltpu.make_async_copy(tab_fu.at[pl.ds(0, HALF), :], rfu, sem1).wait()
        pltpu.make_async_copy(tab_mi.at[pl.ds(0, HALF), :], rmi, sem2).wait()
        pltpu.make_async_copy(tab_fi.at[pl.ds(0, HALF), :], rfi, sem3).wait()
        lax.fori_loop(0, HALF // L, functools.partial(compute, h=h), 0)

    pltpu.sync_copy(outbuf, out.at[pl.ds(base, BPW)])


@functools.partial(jax.jit, static_argnums=())
def _sc_forward(user, item, tab_mu, tab_fu, tab_mi, tab_fi, wpack):
    mesh = plsc.VectorSubcoreMesh(core_axis_name="c", subcore_axis_name="s")
    scratch = (
        pltpu.VMEM((BPW,), jnp.int32),
        pltpu.VMEM((BPW,), jnp.int32),
        pltpu.VMEM((HALF, D), jnp.float32),
        pltpu.VMEM((HALF, D), jnp.float32),
        pltpu.VMEM((HALF, D), jnp.float32),
        pltpu.VMEM((HALF, D), jnp.float32),
        pltpu.VMEM((2, 128), jnp.float32),
        pltpu.VMEM((BPW,), jnp.float32),
        pltpu.SemaphoreType.DMA,
        pltpu.SemaphoreType.DMA,
        pltpu.SemaphoreType.DMA,
        pltpu.SemaphoreType.DMA,
    )
    f = pl.kernel(
        _sc_body,
        out_type=jax.ShapeDtypeStruct((B,), jnp.float32),
        mesh=mesh,
        scratch_types=scratch,
    )
    return f(user, item, tab_mu, tab_fu, tab_mi, tab_fi, wpack)


def kernel(user, item, mlp_user_table, mf_user_table, mlp_item_table,
           mf_item_table, W1, b1, W2, b2, W3, b3, Wp, bp):
    w2d, c2d = _fold(W1, b1, W2, b2, W3, b3, Wp, bp)
    row1 = jnp.concatenate(
        [Wp[32:, 0], jnp.broadcast_to(c2d[0, 0], (64,))])
    wpack = jnp.stack([w2d[:, 0], row1])
    out = _sc_forward(user.astype(jnp.int32), item.astype(jnp.int32),
                      mlp_user_table, mf_user_table, mlp_item_table,
                      mf_item_table, wpack)
    return out.reshape(B, 1)
